# baseline (speedup 1.0000x reference)
import functools

import jax
import jax.numpy as jnp
from jax import lax
from jax.experimental import pallas as pl
from jax.experimental.pallas import tpu as pltpu
from jax.experimental.pallas import tpu_sc as plsc

D = 128
NC = 2
NS = 16
NW = NC * NS
R = 4
K = 4
SEC = 128


def _sc_gather(x, weight):
    batch, fields = x.shape
    nb_per_w = batch // NW
    nsec = nb_per_w // SEC
    cps = SEC // R
    mesh = plsc.VectorSubcoreMesh(core_axis_name="c", subcore_axis_name="s")

    @functools.partial(
        pl.kernel,
        mesh=mesh,
        out_type=jax.ShapeDtypeStruct((batch, fields, D), jnp.float32),
        scratch_types=[
            pltpu.VMEM((SEC, fields), jnp.int32),
            pltpu.VMEM((K, R, fields, D), jnp.float32),
            pltpu.SemaphoreType.DMA,
            pltpu.SemaphoreType.DMA,
        ],
        compiler_params=pltpu.CompilerParams(use_tc_tiling_on_sc=True),
    )
    def k(x_hbm, w_hbm, out_hbm, idx_v, buf_v, gsem, ssem):
        wid = lax.axis_index("s") * NC + lax.axis_index("c")
        base = wid * nb_per_w

        def start_gathers(sec_row0, c, b):
            return [
                pltpu.async_copy(
                    w_hbm.at[idx_v.at[c * R + r]], buf_v.at[b, r], gsem
                )
                for r in range(R)
            ]

        def start_scatter(sec_row0, c, b):
            row0 = sec_row0 + c * R
            return pltpu.async_copy(
                buf_v.at[b], out_hbm.at[pl.ds(row0, R)], ssem
            )

        def wait_scatter(sec_row0, c, b):
            row0 = sec_row0 + c * R
            pltpu.make_async_copy(
                buf_v.at[b], out_hbm.at[pl.ds(row0, R)], ssem
            ).wait()

        def section(s, carry):
            sec_row0 = base + s * SEC
            pltpu.sync_copy(x_hbm.at[pl.ds(sec_row0, SEC)], idx_v)

            gathers = [start_gathers(sec_row0, b, b) for b in range(K)]
            for b in range(K):
                for g in gathers[b]:
                    g.wait()
                start_scatter(sec_row0, b, b)

            def group(g, carry2):
                c0 = g * K
                gathers = []
                for b in range(K):
                    wait_scatter(sec_row0, c0 - K + b, b)
                    gathers.append(start_gathers(sec_row0, c0 + b, b))
                for b in range(K):
                    for gg in gathers[b]:
                        gg.wait()
                    start_scatter(sec_row0, c0 + b, b)
                return carry2

            lax.fori_loop(1, cps // K, group, 0, unroll=False)

            for b in range(K):
                wait_scatter(sec_row0, cps - K + b, b)
            return carry

        lax.fori_loop(0, nsec, section, 0, unroll=False)

    return k(x, weight)


def kernel(x, weight):
    batch, fields = x.shape
    assert batch % (NW * SEC) == 0 and SEC % (R * K) == 0
    return _sc_gather(x.astype(jnp.int32), weight)


# speedup vs baseline: 2.0780x; 2.0780x over previous
import functools

import jax
import jax.numpy as jnp
from jax import lax
from jax.experimental import pallas as pl
from jax.experimental.pallas import tpu as pltpu
from jax.experimental.pallas import tpu_sc as plsc

D = 128
NC = 2
NS = 16
NW = NC * NS
CH = 128
K = 4


def _sc_gather(xt, weight):
    fields, batch = xt.shape
    nb_per_w = batch // NW
    cpf = nb_per_w // CH
    mesh = plsc.VectorSubcoreMesh(core_axis_name="c", subcore_axis_name="s")

    @functools.partial(
        pl.kernel,
        mesh=mesh,
        out_type=jax.ShapeDtypeStruct((fields, batch, D), jnp.float32),
        scratch_types=[
            pltpu.VMEM((fields, nb_per_w), jnp.int32),
            pltpu.VMEM((K, CH, D), jnp.float32),
            pltpu.SemaphoreType.DMA,
            pltpu.SemaphoreType.DMA,
        ],
    )
    def k(xt_hbm, w_hbm, out_hbm, idx_v, buf_v, gsem, ssem):
        wid = lax.axis_index("s") * NC + lax.axis_index("c")
        base = wid * nb_per_w
        pltpu.sync_copy(xt_hbm.at[:, pl.ds(base, nb_per_w)], idx_v)

        def start_gather(j, cc, b):
            return pltpu.async_copy(
                w_hbm.at[idx_v.at[j, pl.ds(cc * CH, CH)]], buf_v.at[b], gsem
            )

        def start_scatter(j, cc, b):
            return pltpu.async_copy(
                buf_v.at[b], out_hbm.at[j, pl.ds(base + cc * CH, CH)], ssem
            )

        def wait_scatter(j, cc, b):
            pltpu.make_async_copy(
                buf_v.at[b], out_hbm.at[j, pl.ds(base + cc * CH, CH)], ssem
            ).wait()

        gathers = [start_gather(0, b, b) for b in range(K)]
        for b in range(K):
            gathers[b].wait()
            start_scatter(0, b, b)

        def field(j, carry):
            gathers = []
            for b in range(K):
                wait_scatter(j - 1, b, b)
                gathers.append(start_gather(j, b, b))
            for b in range(K):
                gathers[b].wait()
                start_scatter(j, b, b)
            return carry

        lax.fori_loop(1, fields, field, 0, unroll=False)

        for b in range(K):
            wait_scatter(fields - 1, b, b)

    return k(xt, weight)


def kernel(x, weight):
    batch, fields = x.shape
    assert batch % (NW * CH * K) == 0 and batch // NW // CH == K
    xt = jnp.transpose(x.astype(jnp.int32))
    out_t = _sc_gather(xt, weight)
    return jnp.transpose(out_t, (1, 0, 2))
